# baseline (speedup 1.0000x reference)
import functools

import jax
import jax.numpy as jnp
from jax import lax
from jax.experimental import pallas as pl
from jax.experimental.pallas import tpu as pltpu
from jax.experimental.pallas import tpu_sc as plsc

B = 16384
L = 50
D = 32
V = 1000000
NC = 2
NS = 16
NW = NC * NS
SAMPLES_PER_CHUNK = 2
IDX_PER_CHUNK = SAMPLES_PER_CHUNK * L
SW = B // NW
CW = SW // SAMPLES_PER_CHUNK
INV_L = 1.0 / L
NBUF = 8

TCOLS = V // 128 + 1
TFULL = TCOLS - 1
TQ, TR = divmod(TCOLS, NW)


def _transpose_body(t_hbm, tail_hbm, out_hbm, in_b, out_b, tail_v, sem_i, sem_o):
    wid = lax.axis_index("s") * NC + lax.axis_index("c")

    lo = wid * TQ + jnp.minimum(wid, TR)
    cnt = TQ + jnp.where(wid < TR, 1, 0)
    hi = lo + cnt

    io16 = lax.broadcasted_iota(jnp.int32, (16,), 0)
    rowp = lax.shift_right_logical(io16, 2)
    colp = (io16 & 3) * 32

    def start_in(t, b):
        pltpu.async_copy(
            t_hbm.at[pl.ds(0, 32), pl.ds(t * 128, 128)],
            in_b.at[b], sem_i.at[b])

    def wait_in(b):
        pltpu.make_async_copy(
            t_hbm.at[pl.ds(0, 32), pl.ds(0, 128)],
            in_b.at[b], sem_i.at[b]).wait()

    def transpose_tilecol(b, ngroups):
        for d in range(32):
            fvec = (io16 + d) & 31
            colv = colp + fvec
            xs = [plsc.load_gather(in_b.at[b], [fvec, io16 + vg * 16])
                  for vg in range(ngroups)]
            for vg in range(ngroups):
                rowv = rowp + (vg * 4)
                plsc.store_scatter(out_b.at[b], [rowv, colv], xs[vg])

    hi_full = jnp.minimum(hi, TFULL)
    n_full = hi_full - lo

    start_in(lo, 0)

    @pl.when(1 < n_full)
    def _():
        start_in(lo + 1, 1)

    @pl.loop(0, n_full, step=3)
    def _(i3):
        for b in range(3):
            i = i3 + b
            t = lo + i

            @pl.when(i < n_full)
            def _():
                @pl.when(i + 2 < n_full)
                def _():
                    start_in(t + 2, (b + 2) % 3)

                wait_in(b)

                @pl.when(i >= 3)
                def _():
                    pltpu.make_async_copy(
                        t_hbm.at[pl.ds(0, 8), pl.ds(0, 128)],
                        out_b.at[b], sem_o.at[b]).wait()

                transpose_tilecol(b, 8)
                pltpu.async_copy(
                    out_b.at[b], out_hbm.at[pl.ds(t * 32, 32)], sem_o.at[b])

    @pl.loop(0, 3)
    def _(b2):
        @pl.when(b2 < jnp.minimum(n_full, 3))
        def _():
            pltpu.make_async_copy(
                t_hbm.at[pl.ds(0, 8), pl.ds(0, 128)],
                out_b.at[b2], sem_o.at[b2]).wait()

    @pl.when(hi == TCOLS)
    def _():
        pltpu.sync_copy(tail_hbm, tail_v)
        for v in range(64):
            out_b[0, v >> 2, pl.ds((v & 3) * 32, 16)] = tail_v[v, pl.ds(0, 16)]
            out_b[0, v >> 2, pl.ds((v & 3) * 32 + 16, 16)] = (
                tail_v[v, pl.ds(16, 16)])
        pltpu.sync_copy(
            out_b.at[0].at[pl.ds(0, 16)],
            out_hbm.at[pl.ds(TFULL * 32, 16)])


def _gather_body(x_hbm, table_hbm, out_hbm, idx_v, rows_b, out_v, sems):
    wid = lax.axis_index("s") * NC + lax.axis_index("c")

    pltpu.sync_copy(x_hbm.at[pl.ds(wid * CW, CW)], idx_v)

    def start(c, b):
        pltpu.async_copy(table_hbm.at[idx_v.at[c]], rows_b.at[b], sems.at[b])

    def wait(b):
        pltpu.make_async_copy(
            table_hbm.at[pl.ds(0, IDX_PER_CHUNK)], rows_b.at[b],
            sems.at[b]).wait()

    def reduce_chunk(b, c):
        rows = rows_b.at[b]
        for s in range(SAMPLES_PER_CHUNK):
            acc0a = jnp.zeros((16,), jnp.float32)
            acc0b = jnp.zeros((16,), jnp.float32)
            acc1a = jnp.zeros((16,), jnp.float32)
            acc1b = jnp.zeros((16,), jnp.float32)
            for r in range(0, L, 2):
                acc0a = acc0a + rows[s * L + r, pl.ds(0, 16)]
                acc1a = acc1a + rows[s * L + r, pl.ds(16, 16)]
                acc0b = acc0b + rows[s * L + r + 1, pl.ds(0, 16)]
                acc1b = acc1b + rows[s * L + r + 1, pl.ds(16, 16)]
            out_v[SAMPLES_PER_CHUNK * c + s, pl.ds(0, 16)] = (
                acc0a + acc0b) * INV_L
            out_v[SAMPLES_PER_CHUNK * c + s, pl.ds(16, 16)] = (
                acc1a + acc1b) * INV_L

    for b in range(NBUF):
        start(b, b)

    @pl.loop(0, CW // NBUF)
    def _(i):
        base = i * NBUF
        for b in range(NBUF):
            c = base + b
            wait(b)
            reduce_chunk(b, c)

            @pl.when(c + NBUF < CW)
            def _():
                start(c + NBUF, b)

    pltpu.sync_copy(out_v, out_hbm.at[pl.ds(wid * SW, SW)])


@jax.jit
def kernel(x, table):
    mesh = plsc.VectorSubcoreMesh(
        core_axis_name="c", subcore_axis_name="s",
        num_cores=NC, num_subcores=NS,
    )

    transpose = pl.kernel(
        _transpose_body,
        out_type=jax.ShapeDtypeStruct((V // 4, 128), jnp.float32),
        mesh=mesh,
        scratch_types=[
            pltpu.VMEM((3, 32, 128), jnp.float32),
            pltpu.VMEM((3, 32, 128), jnp.float32),
            pltpu.VMEM((64, 128), jnp.float32),
            pltpu.SemaphoreType.DMA((3,)),
            pltpu.SemaphoreType.DMA((3,)),
        ],
        compiler_params=pltpu.CompilerParams(
            use_tc_tiling_on_sc=True, needs_layout_passes=False),
    )
    tail_pad = jnp.pad(table[TFULL * 128:], ((0, 0), (0, 128 - D)))
    t128 = transpose(table.T, tail_pad)
    t32 = t128.reshape(V, D)

    x2 = x.reshape(B * L // IDX_PER_CHUNK, IDX_PER_CHUNK).astype(jnp.int32)
    gather = pl.kernel(
        _gather_body,
        out_type=jax.ShapeDtypeStruct((B, D), jnp.float32),
        mesh=mesh,
        scratch_types=[
            pltpu.VMEM((CW, IDX_PER_CHUNK), jnp.int32),
            pltpu.VMEM((NBUF, IDX_PER_CHUNK, D), jnp.float32),
            pltpu.VMEM((SW, D), jnp.float32),
            pltpu.SemaphoreType.DMA((NBUF,)),
        ],
        compiler_params=pltpu.CompilerParams(use_tc_tiling_on_sc=False),
    )
    return gather(x2, t32)


# speedup vs baseline: 1.0543x; 1.0543x over previous
import functools

import jax
import jax.numpy as jnp
from jax import lax
from jax.experimental import pallas as pl
from jax.experimental.pallas import tpu as pltpu
from jax.experimental.pallas import tpu_sc as plsc

B = 16384
L = 50
D = 32
V = 1000000
NC = 2
NS = 16
NW = NC * NS
SAMPLES_PER_CHUNK = 2
IDX_PER_CHUNK = SAMPLES_PER_CHUNK * L
SW = B // NW
CW = SW // SAMPLES_PER_CHUNK
INV_L = 1.0 / L
NBUF = 8

TCOLS = V // 128 + 1
TFULL = TCOLS - 1
TQ, TR = divmod(TCOLS, NW)


def _transpose_body(t_hbm, tail_hbm, out_hbm, in_b, out_b, tail_v, sem_i, sem_o):
    wid = lax.axis_index("s") * NC + lax.axis_index("c")

    lo = wid * TQ + jnp.minimum(wid, TR)
    cnt = TQ + jnp.where(wid < TR, 1, 0)
    hi = lo + cnt

    io16 = lax.broadcasted_iota(jnp.int32, (16,), 0)
    rowp = lax.shift_right_logical(io16, 2)
    colp = (io16 & 3) * 32

    def start_in(t, b):
        pltpu.async_copy(
            t_hbm.at[pl.ds(0, 32), pl.ds(t * 128, 128)],
            in_b.at[b], sem_i.at[b])

    def wait_in(b):
        pltpu.make_async_copy(
            t_hbm.at[pl.ds(0, 32), pl.ds(0, 128)],
            in_b.at[b], sem_i.at[b]).wait()

    def transpose_tilecol(b, ngroups):
        for d in range(32):
            fvec = (io16 + d) & 31
            colv = colp + fvec
            xs = [plsc.load_gather(in_b.at[b], [fvec, io16 + vg * 16])
                  for vg in range(ngroups)]
            for vg in range(ngroups):
                rowv = rowp + (vg * 4)
                plsc.store_scatter(out_b.at[b], [rowv, colv], xs[vg])

    hi_full = jnp.minimum(hi, TFULL)
    n_full = hi_full - lo

    start_in(lo, 0)

    @pl.loop(0, n_full, step=2)
    def _(i2):
        for b in range(2):
            i = i2 + b
            t = lo + i

            @pl.when(i < n_full)
            def _():
                @pl.when(i + 1 < n_full)
                def _():
                    start_in(t + 1, 1 - b)

                wait_in(b)

                @pl.when(i >= 2)
                def _():
                    pltpu.make_async_copy(
                        t_hbm.at[pl.ds(0, 8), pl.ds(0, 128)],
                        out_b.at[b], sem_o.at[b]).wait()

                transpose_tilecol(b, 8)
                pltpu.async_copy(
                    out_b.at[b], out_hbm.at[pl.ds(t * 32, 32)], sem_o.at[b])

    @pl.loop(0, 2)
    def _(b2):
        @pl.when(b2 < jnp.minimum(n_full, 2))
        def _():
            pltpu.make_async_copy(
                t_hbm.at[pl.ds(0, 8), pl.ds(0, 128)],
                out_b.at[b2], sem_o.at[b2]).wait()

    @pl.when(hi == TCOLS)
    def _():
        pltpu.sync_copy(tail_hbm, tail_v)
        for v in range(64):
            out_b[0, v >> 2, pl.ds((v & 3) * 32, 16)] = tail_v[v, pl.ds(0, 16)]
            out_b[0, v >> 2, pl.ds((v & 3) * 32 + 16, 16)] = (
                tail_v[v, pl.ds(16, 16)])
        pltpu.sync_copy(
            out_b.at[0].at[pl.ds(0, 16)],
            out_hbm.at[pl.ds(TFULL * 32, 16)])


def _gather_body(x_hbm, table_hbm, out_hbm, idx_v, rows_b, out_v, sems):
    wid = lax.axis_index("s") * NC + lax.axis_index("c")

    pltpu.sync_copy(x_hbm.at[pl.ds(wid * CW, CW)], idx_v)

    def start(c, b):
        pltpu.async_copy(table_hbm.at[idx_v.at[c]], rows_b.at[b], sems.at[b])

    def wait(b):
        pltpu.make_async_copy(
            table_hbm.at[pl.ds(0, IDX_PER_CHUNK)], rows_b.at[b],
            sems.at[b]).wait()

    def reduce_chunk(b, c):
        rows = rows_b.at[b]
        for s in range(SAMPLES_PER_CHUNK):
            acc0a = jnp.zeros((16,), jnp.float32)
            acc0b = jnp.zeros((16,), jnp.float32)
            acc1a = jnp.zeros((16,), jnp.float32)
            acc1b = jnp.zeros((16,), jnp.float32)
            for r in range(0, L, 2):
                acc0a = acc0a + rows[s * L + r, pl.ds(0, 16)]
                acc1a = acc1a + rows[s * L + r, pl.ds(16, 16)]
                acc0b = acc0b + rows[s * L + r + 1, pl.ds(0, 16)]
                acc1b = acc1b + rows[s * L + r + 1, pl.ds(16, 16)]
            out_v[SAMPLES_PER_CHUNK * c + s, pl.ds(0, 16)] = (
                acc0a + acc0b) * INV_L
            out_v[SAMPLES_PER_CHUNK * c + s, pl.ds(16, 16)] = (
                acc1a + acc1b) * INV_L

    for b in range(NBUF):
        start(b, b)

    @pl.loop(0, CW // NBUF)
    def _(i):
        base = i * NBUF
        for b in range(NBUF):
            c = base + b
            wait(b)
            reduce_chunk(b, c)

            @pl.when(c + NBUF < CW)
            def _():
                start(c + NBUF, b)

    pltpu.sync_copy(out_v, out_hbm.at[pl.ds(wid * SW, SW)])


@jax.jit
def kernel(x, table):
    mesh = plsc.VectorSubcoreMesh(
        core_axis_name="c", subcore_axis_name="s",
        num_cores=NC, num_subcores=NS,
    )

    transpose = pl.kernel(
        _transpose_body,
        out_type=jax.ShapeDtypeStruct((V // 4, 128), jnp.float32),
        mesh=mesh,
        scratch_types=[
            pltpu.VMEM((2, 32, 128), jnp.float32),
            pltpu.VMEM((2, 32, 128), jnp.float32),
            pltpu.VMEM((64, 128), jnp.float32),
            pltpu.SemaphoreType.DMA((2,)),
            pltpu.SemaphoreType.DMA((2,)),
        ],
        compiler_params=pltpu.CompilerParams(
            use_tc_tiling_on_sc=True, needs_layout_passes=False),
    )
    tail_pad = jnp.pad(table[TFULL * 128:], ((0, 0), (0, 128 - D)))
    t128 = transpose(table.T, tail_pad)
    t32 = t128.reshape(V, D)

    x2 = x.reshape(B * L // IDX_PER_CHUNK, IDX_PER_CHUNK).astype(jnp.int32)
    gather = pl.kernel(
        _gather_body,
        out_type=jax.ShapeDtypeStruct((B, D), jnp.float32),
        mesh=mesh,
        scratch_types=[
            pltpu.VMEM((CW, IDX_PER_CHUNK), jnp.int32),
            pltpu.VMEM((NBUF, IDX_PER_CHUNK, D), jnp.float32),
            pltpu.VMEM((SW, D), jnp.float32),
            pltpu.SemaphoreType.DMA((NBUF,)),
        ],
        compiler_params=pltpu.CompilerParams(use_tc_tiling_on_sc=False),
    )
    return gather(x2, t32)
